# baseline (speedup 1.0000x reference)
import functools

import jax
import jax.numpy as jnp
from jax import lax
from jax.experimental import pallas as pl
from jax.experimental.pallas import tpu as pltpu
from jax.experimental.pallas import tpu_sc as plsc

_N_NODES = 100000
_N_EDGES = 3200000
_D_EDGE = 16
_D_NODE = 128

_NC = 2
_NS = 16
_NW = _NC * _NS

_CHUNK = 4096
_ROWS = _CHUNK // 128
_N_FULL = _N_EDGES // _CHUNK
_PER_CORE = _N_FULL // 2
_TAIL = _N_EDGES - _N_FULL * _CHUNK
_TAIL_ROWS = _TAIL // 128
_NBUF = 3
_MAX_ITERS = (_N_FULL - _PER_CORE + _NBUF - 1) // _NBUF
_NODES_PAD = 100096
_GROUPS = _CHUNK // 16


@functools.partial(
    pl.kernel,
    out_type=jax.ShapeDtypeStruct((_NW * _NODES_PAD,), jnp.float32),
    mesh=plsc.VectorSubcoreMesh(
        core_axis_name="c", subcore_axis_name="s",
        num_cores=_NC, num_subcores=_NS,
    ),
    scratch_types=[
        pltpu.VMEM((_NBUF, _CHUNK), jnp.int32),
        pltpu.VMEM((_NBUF, _ROWS, 128), jnp.float32),
        pltpu.VMEM((_NODES_PAD,), jnp.float32),
        pltpu.SemaphoreType.DMA((_NBUF,)),
        pltpu.SemaphoreType.DMA((_NBUF,)),
    ],
    compiler_params=pltpu.CompilerParams(use_tc_tiling_on_sc=False,
                                         needs_layout_passes=False),
)
def _segment_sum_sc(attr_hbm, col_hbm, out_hbm,
                    idx_v, val_v, acc, isem, vsem):
    c = lax.axis_index("c")
    s = lax.axis_index("s")
    wid = c * _NS + s

    zero16 = jnp.zeros((16,), jnp.float32)

    def _zero_body(i, carry):
        acc[pl.ds(i * 16, 16)] = zero16
        return carry

    lax.fori_loop(0, _NODES_PAD // 16, _zero_body, 0)

    n = _PER_CORE + c
    base = c * _PER_CORE

    def _start_load(k, b):
        gid = base + k
        pltpu.async_copy(
            col_hbm.at[pl.ds(pl.multiple_of(gid * _CHUNK, 8), _CHUNK)],
            idx_v.at[b], isem.at[b])
        pltpu.async_copy(
            attr_hbm.at[s, pl.ds(pl.multiple_of(gid * _ROWS, 8), _ROWS)],
            val_v.at[b], vsem.at[b])

    def _wait_load(b):
        pltpu.make_async_copy(col_hbm.at[pl.ds(0, _CHUNK)], idx_v.at[b],
                              isem.at[b]).wait()
        pltpu.make_async_copy(attr_hbm.at[0, pl.ds(0, _ROWS)], val_v.at[b],
                              vsem.at[b]).wait()

    def _load_group(b, g):
        iv = idx_v[b, pl.ds(g * 16, 16)]
        vals = val_v[b, g // 8, pl.ds((g % 8) * 16, 16)]
        return iv, vals

    def _compute(b, groups=_GROUPS):
        pending = [_load_group(b, 0), _load_group(b, 1)]
        for g in range(groups):
            if g + 2 < groups:
                pending.append(_load_group(b, g + 2))
            iv, vals = pending[g]
            plsc.addupdate_scatter(acc, [iv], vals)

    for b in range(_NBUF):
        @pl.when(b < n)
        def _():
            _start_load(b, b)

    def _body(g, carry):
        for b in range(_NBUF):
            k = g * _NBUF + b

            @pl.when(k < n)
            def _():
                _wait_load(b)
                _compute(b)

                @pl.when(k + _NBUF < n)
                def _():
                    _start_load(k + _NBUF, b)
        return carry

    lax.fori_loop(0, _MAX_ITERS, _body, 0)

    @pl.when(c == 1)
    def _():
        pltpu.async_copy(
            col_hbm.at[pl.ds(_N_FULL * _CHUNK, _TAIL)],
            idx_v.at[0, pl.ds(0, _TAIL)], isem.at[0])
        pltpu.async_copy(
            attr_hbm.at[s, pl.ds(_N_FULL * _ROWS, _TAIL_ROWS)],
            val_v.at[0, pl.ds(0, _TAIL_ROWS)], vsem.at[0])
        pltpu.make_async_copy(col_hbm.at[pl.ds(0, _TAIL)],
                              idx_v.at[0, pl.ds(0, _TAIL)], isem.at[0]).wait()
        pltpu.make_async_copy(attr_hbm.at[0, pl.ds(0, _TAIL_ROWS)],
                              val_v.at[0, pl.ds(0, _TAIL_ROWS)],
                              vsem.at[0]).wait()
        _compute(0, groups=_TAIL // 16)

    pltpu.sync_copy(
        acc, out_hbm.at[pl.ds(pl.multiple_of(wid * _NODES_PAD, 8), _NODES_PAD)])


_BLK = 4000


def _mlp_body(x_ref, agg_ref, w1x_ref, w1a_ref, b1_ref, w2_ref, b2_ref,
              out_ref):
    h = jnp.dot(x_ref[...], w1x_ref[...], preferred_element_type=jnp.float32)
    h += jnp.dot(agg_ref[...], w1a_ref[...], preferred_element_type=jnp.float32)
    h = jnp.maximum(h + b1_ref[...], 0.0)
    o = jnp.dot(h, w2_ref[...], preferred_element_type=jnp.float32)
    out_ref[...] = jnp.maximum(o + b2_ref[...], 0.0)


def _mlp_tc(x, agg, w1x, w1a, b1, w2, b2):
    h2 = w2.shape[1]
    grid = _N_NODES // _BLK
    return pl.pallas_call(
        _mlp_body,
        grid=(grid,),
        in_specs=[
            pl.BlockSpec((_BLK, _D_NODE), lambda i: (i, 0)),
            pl.BlockSpec((_BLK, _D_EDGE), lambda i: (i, 0)),
            pl.BlockSpec((_D_NODE, w1x.shape[1]), lambda i: (0, 0)),
            pl.BlockSpec((_D_EDGE, w1a.shape[1]), lambda i: (0, 0)),
            pl.BlockSpec((1, b1.shape[1]), lambda i: (0, 0)),
            pl.BlockSpec((w2.shape[0], h2), lambda i: (0, 0)),
            pl.BlockSpec((1, b2.shape[1]), lambda i: (0, 0)),
        ],
        out_specs=pl.BlockSpec((_BLK, h2), lambda i: (i, 0)),
        out_shape=jax.ShapeDtypeStruct((_N_NODES, h2), jnp.float32),
    )(x, agg, w1x, w1a, b1, w2, b2)


def kernel(x, edge_index, edge_attr, W1, b1, W2, b2):
    attr_t = edge_attr.T.reshape(_D_EDGE, _N_EDGES // 128, 128)
    col = edge_index[1].astype(jnp.int32).reshape(-1)
    partials = _segment_sum_sc(attr_t, col)
    p = partials.reshape(_NC, _NS, _NODES_PAD)
    agg = (p[0] + p[1]).T
    return _mlp_tc(
        x, agg[:_N_NODES],
        W1[:_D_NODE], W1[_D_NODE:],
        b1.reshape(1, -1), W2, b2.reshape(1, -1),
    )


# speedup vs baseline: 1.2411x; 1.2411x over previous
import functools

import jax
import jax.numpy as jnp
from jax import lax
from jax.experimental import pallas as pl
from jax.experimental.pallas import tpu as pltpu
from jax.experimental.pallas import tpu_sc as plsc

_N_NODES = 100000
_N_EDGES = 3200000
_D_EDGE = 16
_D_NODE = 128

_NC = 2
_NS = 16
_NW = _NC * _NS

_CHUNK = 2048
_ROWS = _CHUNK // 128
_N_FULL = _N_EDGES // _CHUNK
_PER_CORE = _N_FULL // 2
_TAIL = _N_EDGES - _N_FULL * _CHUNK
_TAIL_ROWS = _TAIL // 128
_NBUF = 3
_MAX_ITERS = (_N_FULL - _PER_CORE + _NBUF - 1) // _NBUF
_NODES_PAD = 100096
_GROUPS = _CHUNK // 16


@functools.partial(
    pl.kernel,
    out_type=jax.ShapeDtypeStruct((_NW * _NODES_PAD,), jnp.float32),
    mesh=plsc.VectorSubcoreMesh(
        core_axis_name="c", subcore_axis_name="s",
        num_cores=_NC, num_subcores=_NS,
    ),
    scratch_types=[
        pltpu.VMEM((_NBUF, _CHUNK), jnp.int32),
        pltpu.VMEM((_NBUF, _ROWS, 128), jnp.float32),
        pltpu.VMEM((_NODES_PAD,), jnp.float32),
        pltpu.SemaphoreType.DMA((_NBUF,)),
        pltpu.SemaphoreType.DMA((_NBUF,)),
    ],
    compiler_params=pltpu.CompilerParams(use_tc_tiling_on_sc=False,
                                         needs_layout_passes=False),
)
def _segment_sum_sc(attr_hbm, col_hbm, out_hbm,
                    idx_v, val_v, acc, isem, vsem):
    c = lax.axis_index("c")
    s = lax.axis_index("s")
    wid = c * _NS + s

    zero16 = jnp.zeros((16,), jnp.float32)

    def _zero_body(i, carry):
        acc[pl.ds(i * 16, 16)] = zero16
        return carry

    lax.fori_loop(0, _NODES_PAD // 16, _zero_body, 0)

    n = _PER_CORE + c * (_N_FULL - 2 * _PER_CORE)
    base = c * _PER_CORE

    def _start_load(k, b):
        gid = base + k
        pltpu.async_copy(
            col_hbm.at[pl.ds(pl.multiple_of(gid * _CHUNK, 8), _CHUNK)],
            idx_v.at[b], isem.at[b])
        pltpu.async_copy(
            attr_hbm.at[s, pl.ds(pl.multiple_of(gid * _ROWS, 8), _ROWS)],
            val_v.at[b], vsem.at[b])

    def _wait_load(b):
        pltpu.make_async_copy(col_hbm.at[pl.ds(0, _CHUNK)], idx_v.at[b],
                              isem.at[b]).wait()
        pltpu.make_async_copy(attr_hbm.at[0, pl.ds(0, _ROWS)], val_v.at[b],
                              vsem.at[b]).wait()

    def _load_group(b, g):
        iv = idx_v[b, pl.ds(g * 16, 16)]
        vals = val_v[b, g // 8, pl.ds((g % 8) * 16, 16)]
        return iv, vals

    def _compute(b, groups=_GROUPS):
        pending = [_load_group(b, 0), _load_group(b, 1)]
        for g in range(groups):
            if g + 2 < groups:
                pending.append(_load_group(b, g + 2))
            iv, vals = pending[g]
            plsc.addupdate_scatter(acc, [iv], vals)

    for b in range(_NBUF):
        @pl.when(b < n)
        def _():
            _start_load(b, b)

    def _body(g, carry):
        for b in range(_NBUF):
            k = g * _NBUF + b

            @pl.when(k < n)
            def _():
                _wait_load(b)
                _compute(b)

                @pl.when(k + _NBUF < n)
                def _():
                    _start_load(k + _NBUF, b)
        return carry

    lax.fori_loop(0, _MAX_ITERS, _body, 0)

    @pl.when(c == 1)
    def _():
        pltpu.async_copy(
            col_hbm.at[pl.ds(_N_FULL * _CHUNK, _TAIL)],
            idx_v.at[0, pl.ds(0, _TAIL)], isem.at[0])
        pltpu.async_copy(
            attr_hbm.at[s, pl.ds(_N_FULL * _ROWS, _TAIL_ROWS)],
            val_v.at[0, pl.ds(0, _TAIL_ROWS)], vsem.at[0])
        pltpu.make_async_copy(col_hbm.at[pl.ds(0, _TAIL)],
                              idx_v.at[0, pl.ds(0, _TAIL)], isem.at[0]).wait()
        pltpu.make_async_copy(attr_hbm.at[0, pl.ds(0, _TAIL_ROWS)],
                              val_v.at[0, pl.ds(0, _TAIL_ROWS)],
                              vsem.at[0]).wait()
        _compute(0, groups=_TAIL // 16)

    pltpu.sync_copy(
        acc, out_hbm.at[pl.ds(pl.multiple_of(wid * _NODES_PAD, 8), _NODES_PAD)])


_BLK = 4000


def _mlp_body(x_ref, agg_ref, w1x_ref, w1a_ref, b1_ref, w2_ref, b2_ref,
              out_ref):
    h = jnp.dot(x_ref[...], w1x_ref[...], preferred_element_type=jnp.float32)
    h += jnp.dot(agg_ref[...], w1a_ref[...], preferred_element_type=jnp.float32)
    h = jnp.maximum(h + b1_ref[...], 0.0)
    o = jnp.dot(h, w2_ref[...], preferred_element_type=jnp.float32)
    out_ref[...] = jnp.maximum(o + b2_ref[...], 0.0)


def _mlp_tc(x, agg, w1x, w1a, b1, w2, b2):
    h2 = w2.shape[1]
    grid = _N_NODES // _BLK
    return pl.pallas_call(
        _mlp_body,
        grid=(grid,),
        in_specs=[
            pl.BlockSpec((_BLK, _D_NODE), lambda i: (i, 0)),
            pl.BlockSpec((_BLK, _D_EDGE), lambda i: (i, 0)),
            pl.BlockSpec((_D_NODE, w1x.shape[1]), lambda i: (0, 0)),
            pl.BlockSpec((_D_EDGE, w1a.shape[1]), lambda i: (0, 0)),
            pl.BlockSpec((1, b1.shape[1]), lambda i: (0, 0)),
            pl.BlockSpec((w2.shape[0], h2), lambda i: (0, 0)),
            pl.BlockSpec((1, b2.shape[1]), lambda i: (0, 0)),
        ],
        out_specs=pl.BlockSpec((_BLK, h2), lambda i: (i, 0)),
        out_shape=jax.ShapeDtypeStruct((_N_NODES, h2), jnp.float32),
    )(x, agg, w1x, w1a, b1, w2, b2)


def kernel(x, edge_index, edge_attr, W1, b1, W2, b2):
    attr_t = edge_attr.T.reshape(_D_EDGE, _N_EDGES // 128, 128)
    col = edge_index[1].astype(jnp.int32).reshape(-1)
    partials = _segment_sum_sc(attr_t, col)
    p = partials.reshape(_NC, _NS, _NODES_PAD)
    agg = (p[0] + p[1]).T
    return _mlp_tc(
        x, agg[:_N_NODES],
        W1[:_D_NODE], W1[_D_NODE:],
        b1.reshape(1, -1), W2, b2.reshape(1, -1),
    )
